# baseline (speedup 1.0000x reference)
import functools

import jax
import jax.numpy as jnp
from jax import lax
from jax.experimental import pallas as pl
from jax.experimental.pallas import tpu as pltpu
from jax.experimental.pallas import tpu_sc as plsc

_MAX_LEN = 8192
_HIDDEN = 128

_INFO = plsc.get_sparse_core_info()
_NC = _INFO.num_cores
_NS = _INFO.num_subcores
_NW = _NC * _NS
_ROWS_PER_W = _MAX_LEN // _NW


def _make_copy():
    mesh = plsc.VectorSubcoreMesh(core_axis_name="c", subcore_axis_name="s")

    @functools.partial(
        pl.kernel,
        mesh=mesh,
        out_type=jax.ShapeDtypeStruct((_MAX_LEN, _HIDDEN), jnp.float32),
    )
    def k(table_hbm, out_hbm):
        wid = lax.axis_index("s") * _NC + lax.axis_index("c")
        base = wid * _ROWS_PER_W
        pltpu.sync_copy(table_hbm.at[pl.ds(base, _ROWS_PER_W)],
                        out_hbm.at[pl.ds(base, _ROWS_PER_W)])

    return k


_copy = _make_copy()


def kernel(x, emb_table):
    seq_len = x.shape[1]
    out = _copy(emb_table)
    return out[None, :seq_len]


# speedup vs baseline: 6.5314x; 6.5314x over previous
import functools

import jax
import jax.numpy as jnp
from jax import lax
from jax.experimental import pallas as pl
from jax.experimental.pallas import tpu as pltpu
from jax.experimental.pallas import tpu_sc as plsc

_MAX_LEN = 8192
_HIDDEN = 128

_INFO = plsc.get_sparse_core_info()
_NC = _INFO.num_cores
_NS = _INFO.num_subcores
_NW = _NC * _NS
_ROWS_PER_W = _MAX_LEN // _NW


def _make_copy():
    mesh = plsc.VectorSubcoreMesh(core_axis_name="c", subcore_axis_name="s")

    @functools.partial(
        pl.kernel,
        mesh=mesh,
        out_type=jax.ShapeDtypeStruct((_MAX_LEN, _HIDDEN), jnp.float32),
        scratch_types=[pltpu.VMEM((_ROWS_PER_W, _HIDDEN), jnp.float32)],
    )
    def k(table_hbm, out_hbm, buf):
        wid = lax.axis_index("s") * _NC + lax.axis_index("c")
        base = wid * _ROWS_PER_W
        pltpu.sync_copy(table_hbm.at[pl.ds(base, _ROWS_PER_W)], buf)
        pltpu.sync_copy(buf, out_hbm.at[pl.ds(base, _ROWS_PER_W)])

    return k


_copy = _make_copy()


def kernel(x, emb_table):
    seq_len = x.shape[1]
    out = _copy(emb_table)
    return out[None, :seq_len]


# speedup vs baseline: 6.5595x; 1.0043x over previous
import functools

import jax
import jax.numpy as jnp
from jax import lax
from jax.experimental import pallas as pl
from jax.experimental.pallas import tpu as pltpu
from jax.experimental.pallas import tpu_sc as plsc

_MAX_LEN = 8192
_HIDDEN = 128

_INFO = plsc.get_sparse_core_info()
_NC = _INFO.num_cores
_NS = _INFO.num_subcores
_NW = _NC * _NS
_ROWS_PER_W = _MAX_LEN // _NW


def _make_copy():
    mesh = plsc.VectorSubcoreMesh(core_axis_name="c", subcore_axis_name="s")

    n_chunks = 4
    chunk = _ROWS_PER_W // n_chunks

    @functools.partial(
        pl.kernel,
        mesh=mesh,
        out_type=jax.ShapeDtypeStruct((_MAX_LEN, _HIDDEN), jnp.float32),
        scratch_types=(
            [pltpu.VMEM((_ROWS_PER_W, _HIDDEN), jnp.float32)]
            + [pltpu.SemaphoreType.DMA] * (2 * n_chunks)
        ),
    )
    def k(table_hbm, out_hbm, buf, *sems):
        rsems, wsems = sems[:n_chunks], sems[n_chunks:]
        wid = lax.axis_index("s") * _NC + lax.axis_index("c")
        base = wid * _ROWS_PER_W
        reads = []
        for i in range(n_chunks):
            reads.append(pltpu.async_copy(
                table_hbm.at[pl.ds(base + i * chunk, chunk)],
                buf.at[pl.ds(i * chunk, chunk)], rsems[i]))
        writes = []
        for i in range(n_chunks):
            reads[i].wait()
            writes.append(pltpu.async_copy(
                buf.at[pl.ds(i * chunk, chunk)],
                out_hbm.at[pl.ds(base + i * chunk, chunk)], wsems[i]))
        for w in writes:
            w.wait()

    return k


_copy = _make_copy()


def kernel(x, emb_table):
    seq_len = x.shape[1]
    out = _copy(emb_table)
    return out[None, :seq_len]


# speedup vs baseline: 6.5677x; 1.0013x over previous
import functools

import jax
import jax.numpy as jnp
from jax import lax
from jax.experimental import pallas as pl
from jax.experimental.pallas import tpu as pltpu
from jax.experimental.pallas import tpu_sc as plsc

_MAX_LEN = 8192
_HIDDEN = 128

_INFO = plsc.get_sparse_core_info()
_NC = _INFO.num_cores
_ROWS_PER_C = _MAX_LEN // _NC


def _make_copy():
    mesh = plsc.ScalarSubcoreMesh(axis_name="c", num_cores=_NC)
    n_chunks = 8
    chunk = _ROWS_PER_C // n_chunks

    @functools.partial(
        pl.kernel,
        mesh=mesh,
        out_type=jax.ShapeDtypeStruct((_MAX_LEN, _HIDDEN), jnp.float32),
        scratch_types=(
            [pltpu.VMEM_SHARED((_ROWS_PER_C, _HIDDEN), jnp.float32)]
            + [pltpu.SemaphoreType.DMA] * (2 * n_chunks)
        ),
    )
    def k(table_hbm, out_hbm, buf, *sems):
        rsems, wsems = sems[:n_chunks], sems[n_chunks:]
        base = lax.axis_index("c") * _ROWS_PER_C
        reads = []
        for i in range(n_chunks):
            reads.append(pltpu.async_copy(
                table_hbm.at[pl.ds(base + i * chunk, chunk)],
                buf.at[pl.ds(i * chunk, chunk)], rsems[i]))
        writes = []
        for i in range(n_chunks):
            reads[i].wait()
            writes.append(pltpu.async_copy(
                buf.at[pl.ds(i * chunk, chunk)],
                out_hbm.at[pl.ds(base + i * chunk, chunk)], wsems[i]))
        for w in writes:
            w.wait()

    return k


_copy = _make_copy()


def kernel(x, emb_table):
    seq_len = x.shape[1]
    out = _copy(emb_table)
    return out[None, :seq_len]


# speedup vs baseline: 13.5591x; 2.0645x over previous
import jax
import jax.numpy as jnp
from jax.experimental import pallas as pl

_MAX_LEN = 8192
_HIDDEN = 128
_BLK = 512


def _body(in_ref, out_ref):
    out_ref[...] = in_ref[...]


def kernel(x, emb_table):
    seq_len = x.shape[1]
    out = pl.pallas_call(
        _body,
        grid=(_MAX_LEN // _BLK,),
        in_specs=[pl.BlockSpec((_BLK, _HIDDEN), lambda i: (i, 0))],
        out_specs=pl.BlockSpec((_BLK, _HIDDEN), lambda i: (i, 0)),
        out_shape=jax.ShapeDtypeStruct((_MAX_LEN, _HIDDEN), jnp.float32),
    )(emb_table)
    return out[None, :seq_len]


# speedup vs baseline: 29.6742x; 2.1885x over previous
import jax
import jax.numpy as jnp
from jax.experimental import pallas as pl

_MAX_LEN = 8192
_HIDDEN = 128
_BLK = 2048


def _body(in_ref, out_ref):
    out_ref[...] = in_ref[...]


def kernel(x, emb_table):
    seq_len = x.shape[1]
    out = pl.pallas_call(
        _body,
        grid=(_MAX_LEN // _BLK,),
        in_specs=[pl.BlockSpec((_BLK, _HIDDEN), lambda i: (i, 0))],
        out_specs=pl.BlockSpec((_BLK, _HIDDEN), lambda i: (i, 0)),
        out_shape=jax.ShapeDtypeStruct((_MAX_LEN, _HIDDEN), jnp.float32),
    )(emb_table)
    return out[None, :seq_len]


# speedup vs baseline: 38.6872x; 1.3037x over previous
import jax
import jax.numpy as jnp
from jax.experimental import pallas as pl

_MAX_LEN = 8192
_HIDDEN = 128
_BLK = 4096


def _body(in_ref, out_ref):
    out_ref[...] = in_ref[...]


def kernel(x, emb_table):
    seq_len = x.shape[1]
    out = pl.pallas_call(
        _body,
        grid=(_MAX_LEN // _BLK,),
        in_specs=[pl.BlockSpec((_BLK, _HIDDEN), lambda i: (i, 0))],
        out_specs=pl.BlockSpec((_BLK, _HIDDEN), lambda i: (i, 0)),
        out_shape=jax.ShapeDtypeStruct((_MAX_LEN, _HIDDEN), jnp.float32),
    )(emb_table)
    return out[None, :seq_len]
